# baseline (speedup 1.0000x reference)
import functools

import jax
import jax.numpy as jnp
from jax import lax
from jax.experimental import pallas as pl
from jax.experimental.pallas import tpu as pltpu
from jax.experimental.pallas import tpu_sc as plsc

_B = 16384
_D = 32
_DP = _D // 2
_N = 1000000
_NC = 2
_NS = 16
_NW = _NC * _NS
_PW = _B // _NW
_K = 16
_NB = _PW // _K


@functools.partial(
    pl.kernel,
    mesh=plsc.VectorSubcoreMesh(core_axis_name="c", subcore_axis_name="s"),
    out_type=jax.ShapeDtypeStruct((2 * _D, _B), jnp.float32),
    compiler_params=pltpu.CompilerParams(
        use_tc_tiling_on_sc=True, needs_layout_passes=False),
    scratch_types=[
        pltpu.VMEM((_PW,), jnp.int32),
        pltpu.VMEM((_PW,), jnp.int32),
        pltpu.VMEM((_K, _DP, 128), jnp.int32),
        pltpu.VMEM((_D, _PW), jnp.float32),
        pltpu.VMEM((_D, _PW), jnp.float32),
    ] + [pltpu.SemaphoreType.DMA] * _K,
)
def _edge_gather(src_hbm, dst_hbm, table_t_hbm, out_hbm,
                 idx_s, idx_d, win, cols_s, cols_d, *sems):
    wid = lax.axis_index("s") * _NC + lax.axis_index("c")
    pltpu.sync_copy(src_hbm.at[wid], idx_s)
    pltpu.sync_copy(dst_hbm.at[wid], idx_d)

    p16 = lax.iota(jnp.int32, 16)
    d_even = p16 * 2
    d_odd = d_even + 1

    def enqueue(i, j):
        t0 = pl.multiple_of((i >> 7) * 128, 128)
        pltpu.async_copy(
            table_t_hbm.at[:, pl.ds(t0, 128)], win.at[j], sems[j])

    def extract(cols_ref, i, e, j):
        pltpu.make_async_copy(
            table_t_hbm.at[:, pl.ds(0, 128)], win.at[j], sems[j]).wait()
        j16 = jnp.full((16,), j, jnp.int32)
        l16 = jnp.full((16,), i & 127, jnp.int32)
        e16 = jnp.full((16,), e, jnp.int32)
        w = plsc.load_gather(win, [j16, p16, l16])
        v_even = plsc.bitcast(w << 16, jnp.float32)
        v_odd = plsc.bitcast(w & jnp.int32(-65536), jnp.float32)
        plsc.store_scatter(cols_ref, [d_even, e16], v_even)
        plsc.store_scatter(cols_ref, [d_odd, e16], v_odd)

    def stream(idx_ref, cols_ref):
        iv0 = idx_ref[pl.ds(0, _K)]
        for j in range(_K):
            enqueue(iv0[j], j)

        def round_(b, carry):
            iv_prev = idx_ref[pl.ds((b - 1) * _K, _K)]
            iv_cur = idx_ref[pl.ds(b * _K, _K)]
            for j in range(_K):
                extract(cols_ref, iv_prev[j], (b - 1) * _K + j, j)
                enqueue(iv_cur[j], j)
            return carry

        lax.fori_loop(1, _NB, round_, 0)
        iv_last = idx_ref[pl.ds((_NB - 1) * _K, _K)]
        for j in range(_K):
            extract(cols_ref, iv_last[j], (_NB - 1) * _K + j, j)

    stream(idx_s, cols_s)
    stream(idx_d, cols_d)

    base = wid * _PW
    pltpu.sync_copy(cols_s, out_hbm.at[pl.ds(0, _D), pl.ds(base, _PW)])
    pltpu.sync_copy(cols_d, out_hbm.at[pl.ds(_D, _D), pl.ds(base, _PW)])


def kernel(source_node_input, destination_node_input, table):
    src = source_node_input.reshape(_NW, _PW)
    dst = destination_node_input.reshape(_NW, _PW)
    packed = lax.bitcast_convert_type(
        table.astype(jnp.bfloat16).reshape(_N, _DP, 2), jnp.int32)
    out_t = _edge_gather(src, dst, packed.T)
    return out_t.T


# speedup vs baseline: 3.3020x; 3.3020x over previous
import functools

import jax
import jax.numpy as jnp
from jax import lax
from jax.experimental import pallas as pl
from jax.experimental.pallas import tpu as pltpu
from jax.experimental.pallas import tpu_sc as plsc

_B = 16384
_D = 32
_NC = 2
_NS = 16
_NW = _NC * _NS
_PW = _B // _NW
_K = 16
_NB = _PW // _K


@functools.partial(
    pl.kernel,
    mesh=plsc.VectorSubcoreMesh(core_axis_name="c", subcore_axis_name="s"),
    out_type=jax.ShapeDtypeStruct((2 * _D, _B), jnp.float32),
    compiler_params=pltpu.CompilerParams(
        use_tc_tiling_on_sc=True, needs_layout_passes=False),
    scratch_types=[
        pltpu.VMEM((_PW,), jnp.int32),
        pltpu.VMEM((_PW,), jnp.int32),
        pltpu.VMEM((_K, _D, 128), jnp.float32),
        pltpu.VMEM((_D, _PW), jnp.float32),
        pltpu.VMEM((_D, _PW), jnp.float32),
    ] + [pltpu.SemaphoreType.DMA] * _K,
)
def _edge_gather(src_hbm, dst_hbm, table_t_hbm, out_hbm,
                 idx_s, idx_d, win, cols_s, cols_d, *sems):
    wid = lax.axis_index("s") * _NC + lax.axis_index("c")
    pltpu.sync_copy(src_hbm.at[wid], idx_s)
    pltpu.sync_copy(dst_hbm.at[wid], idx_d)

    d_lo = lax.iota(jnp.int32, 16)
    d_hi = d_lo + 16

    def enqueue(i, j):
        t0 = pl.multiple_of((i >> 7) * 128, 128)
        pltpu.async_copy(
            table_t_hbm.at[:, pl.ds(t0, 128)], win.at[j], sems[j])

    def extract(cols_ref, i, e, j):
        pltpu.make_async_copy(
            table_t_hbm.at[:, pl.ds(0, 128)], win.at[j], sems[j]).wait()
        j16 = jnp.full((16,), j, jnp.int32)
        l16 = jnp.full((16,), i & 127, jnp.int32)
        e16 = jnp.full((16,), e, jnp.int32)
        v_lo = plsc.load_gather(win, [j16, d_lo, l16])
        v_hi = plsc.load_gather(win, [j16, d_hi, l16])
        plsc.store_scatter(cols_ref, [d_lo, e16], v_lo)
        plsc.store_scatter(cols_ref, [d_hi, e16], v_hi)

    def stream(idx_ref, cols_ref):
        iv0 = idx_ref[pl.ds(0, _K)]
        for j in range(_K):
            enqueue(iv0[j], j)

        def round_(b, carry):
            iv_prev = idx_ref[pl.ds((b - 1) * _K, _K)]
            iv_cur = idx_ref[pl.ds(b * _K, _K)]
            for j in range(_K):
                extract(cols_ref, iv_prev[j], (b - 1) * _K + j, j)
                enqueue(iv_cur[j], j)
            return carry

        lax.fori_loop(1, _NB, round_, 0)
        iv_last = idx_ref[pl.ds((_NB - 1) * _K, _K)]
        for j in range(_K):
            extract(cols_ref, iv_last[j], (_NB - 1) * _K + j, j)

    stream(idx_s, cols_s)
    stream(idx_d, cols_d)

    base = wid * _PW
    pltpu.sync_copy(cols_s, out_hbm.at[pl.ds(0, _D), pl.ds(base, _PW)])
    pltpu.sync_copy(cols_d, out_hbm.at[pl.ds(_D, _D), pl.ds(base, _PW)])


def kernel(source_node_input, destination_node_input, table):
    src = source_node_input.reshape(_NW, _PW)
    dst = destination_node_input.reshape(_NW, _PW)
    out_t = _edge_gather(src, dst, table.T)
    return out_t.T
